# baseline (speedup 1.0000x reference)
import functools

import jax
import jax.numpy as jnp
from jax import lax
from jax.experimental import pallas as pl
from jax.experimental.pallas import tpu as pltpu
from jax.experimental.pallas import tpu_sc as plsc

_NUM_ITEMS = 256
_NUM_NESTS = 16
_NUM_NODES = 1 + _NUM_NESTS + _NUM_ITEMS
_B = 512
_S = 20
_NC = 2
_NS = 16
_NW = _NC * _NS
_ROWS = _B // _NW
_L = 16


def _ln(x):
    bits = lax.bitcast_convert_type(x, jnp.int32)
    y = bits.astype(jnp.float32) * jnp.float32(8.2629582e-08) - jnp.float32(87.98997156)
    for _ in range(2):
        y = y + x * jnp.exp(-y) - jnp.float32(1.0)
    return y


def _nested_logit_body(x_hbm, len_hbm, w_hbm, out_hbm,
                       x_v, len_v, w_v, out_v, mark_i, mark_n, acc, sems):
    wid = lax.axis_index("s") * _NC + lax.axis_index("c")
    base = wid * _ROWS
    col = (wid % 8) * _ROWS
    cp_x = pltpu.async_copy(
        x_hbm.at[:, pl.ds(pl.multiple_of((wid // 8) * 128, 128), 128)],
        x_v, sems.at[0])
    cp_l = pltpu.async_copy(len_hbm.at[pl.ds(base, _ROWS)], len_v, sems.at[1])
    cp_w = pltpu.async_copy(w_hbm, w_v, sems.at[2])
    cp_x.wait()
    cp_l.wait()
    cp_w.wait()

    lanes = lax.iota(jnp.int32, _L)

    seg = w_v[pl.ds(_NUM_ITEMS, _L)]
    w_v[pl.ds(_NUM_ITEMS, _L)] = jnp.where(lanes == 0, jnp.float32(0.0), seg)

    def _pos(s):
        xs = plsc.load_gather(x_v, [jnp.broadcast_to(s, (_L,)), col + lanes])
        gs = jnp.right_shift(xs, 4)
        return xs, gs

    neg_inf = jnp.float32(-jnp.inf)

    def _phase_a(j, carry):
        c, cn = carry
        for s in (2 * j, 2 * j + 1):
            xs, gs = _pos(s)
            sv = jnp.broadcast_to(s, (_L,))
            plsc.store_scatter(mark_i, [xs, lanes], sv)
            plsc.store_scatter(mark_n, [gs, lanes], sv)
            u = plsc.load_gather(w_v, [xs + 17])
            nu = plsc.load_gather(w_v, [gs + 1])
            c = jnp.maximum(c, u)
            cn = jnp.maximum(cn, nu)
        return c, cn

    c, cn = lax.fori_loop(
        0, _S // 2, _phase_a,
        (jnp.full((_L,), neg_inf), jnp.full((_L,), neg_inf)))

    zero = jnp.zeros((_L,), jnp.float32)

    def _init_acc(g, carry):
        acc[g, :] = zero
        return carry

    lax.fori_loop(0, _NUM_NESTS, _init_acc, 0)

    def _phase_b(j, sr):
        for s in (2 * j, 2 * j + 1):
            xs, gs = _pos(s)
            u = plsc.load_gather(w_v, [xs + 17])
            nu = plsc.load_gather(w_v, [gs + 1])
            fi = plsc.load_gather(mark_i, [xs, lanes]) == s
            fn = plsc.load_gather(mark_n, [gs, lanes]) == s
            e = jnp.where(fi, jnp.exp(u - c), jnp.float32(0.0))
            plsc.addupdate_scatter(acc, [gs, lanes], e)
            sr = sr + jnp.where(fn, jnp.exp(nu - cn), jnp.float32(0.0))
        return sr

    sr = lax.fori_loop(0, _S // 2, _phase_b, zero)

    ccn = c + cn
    lens = len_v[:]

    def _phase_c(j, carry):
        for s in (2 * j, 2 * j + 1):
            xs, gs = _pos(s)
            u = plsc.load_gather(w_v, [xs + 17])
            nu = plsc.load_gather(w_v, [gs + 1])
            a = plsc.load_gather(acc, [gs, lanes])
            val = u + nu - ccn - _ln(a * sr)
            val = jnp.where(lens > s, val, neg_inf)
            plsc.store_scatter(out_v, [jnp.broadcast_to(s, (_L,)), lanes], val)
        return carry

    lax.fori_loop(0, _S // 2, _phase_c, 0)

    cps = [pltpu.async_copy(out_v.at[s], out_hbm.at[pl.ds(s * _B + base, _ROWS)],
                            sems.at[3]) for s in range(_S)]
    for cp in cps:
        cp.wait()


@functools.cache
def _build_sc_kernel():
    mesh = plsc.VectorSubcoreMesh(core_axis_name="c", subcore_axis_name="s")
    return pl.kernel(
        _nested_logit_body,
        mesh=mesh,
        compiler_params=pltpu.CompilerParams(needs_layout_passes=False),
        out_type=jax.ShapeDtypeStruct((_B * _S,), jnp.float32),
        scratch_types=[
            pltpu.VMEM((_S, 128), jnp.int32),
            pltpu.VMEM((_ROWS,), jnp.int32),
            pltpu.VMEM((_NUM_NODES + 1,), jnp.float32),
            pltpu.VMEM((_S, _ROWS), jnp.float32),
            pltpu.VMEM((_NUM_ITEMS, _L), jnp.int32),
            pltpu.VMEM((_NUM_NESTS, _L), jnp.int32),
            pltpu.VMEM((_NUM_NESTS, _L), jnp.float32),
            pltpu.SemaphoreType.DMA((4,)),
        ],
    )


def kernel(x, x_lengths, weight, leaf_ancestor_matrix, inf_adjacency_matrix):
    del leaf_ancestor_matrix, inf_adjacency_matrix
    Bn, Sn = x.shape
    out_t = _build_sc_kernel()(x.T, x_lengths, weight.reshape(_NUM_NODES + 1))
    return out_t.reshape(Sn, Bn).T.reshape(Bn, Sn, 1)


# speedup vs baseline: 1.0098x; 1.0098x over previous
import functools

import jax
import jax.numpy as jnp
from jax import lax
from jax.experimental import pallas as pl
from jax.experimental.pallas import tpu as pltpu
from jax.experimental.pallas import tpu_sc as plsc

_NUM_ITEMS = 256
_NUM_NESTS = 16
_NUM_NODES = 1 + _NUM_NESTS + _NUM_ITEMS
_B = 512
_S = 20
_NC = 2
_NS = 16
_NW = _NC * _NS
_ROWS = _B // _NW
_L = 16


def _ln(x):
    bits = lax.bitcast_convert_type(x, jnp.int32)
    y = bits.astype(jnp.float32) * jnp.float32(8.2629582e-08) - jnp.float32(87.98997156)
    for _ in range(2):
        y = y + x * jnp.exp(-y) - jnp.float32(1.0)
    return y


def _nested_logit_body(x_hbm, len_hbm, w_hbm, out_hbm,
                       x_v, len_v, w_v, out_v, mark_i, mark_n, acc, sems):
    wid = lax.axis_index("s") * _NC + lax.axis_index("c")
    base = wid * _ROWS
    col = (wid % 8) * _ROWS
    cp_x = pltpu.async_copy(
        x_hbm.at[:, pl.ds(pl.multiple_of((wid // 8) * 128, 128), 128)],
        x_v, sems.at[0])
    cp_l = pltpu.async_copy(len_hbm.at[pl.ds(base, _ROWS)], len_v, sems.at[1])
    cp_w = pltpu.async_copy(w_hbm, w_v, sems.at[2])
    cp_x.wait()
    cp_l.wait()
    cp_w.wait()

    lanes = lax.iota(jnp.int32, _L)

    seg = w_v[pl.ds(_NUM_ITEMS, _L)]
    w_v[pl.ds(_NUM_ITEMS, _L)] = jnp.where(lanes == 0, jnp.float32(0.0), seg)

    def _pos(s):
        xs = plsc.load_gather(x_v, [jnp.broadcast_to(s, (_L,)), col + lanes])
        gs = jnp.right_shift(xs, 4)
        return xs, gs

    neg_inf = jnp.float32(-jnp.inf)

    def _phase_a(s, carry):
        c, cn = carry
        xs, gs = _pos(s)
        sv = jnp.broadcast_to(s, (_L,))
        plsc.store_scatter(mark_i, [xs, lanes], sv)
        plsc.store_scatter(mark_n, [gs, lanes], sv)
        u = plsc.load_gather(w_v, [xs + 17])
        nu = plsc.load_gather(w_v, [gs + 1])
        return jnp.maximum(c, u), jnp.maximum(cn, nu)

    c, cn = lax.fori_loop(
        0, _S, _phase_a,
        (jnp.full((_L,), neg_inf), jnp.full((_L,), neg_inf)))

    zero = jnp.zeros((_L,), jnp.float32)

    def _init_acc(g, carry):
        acc[g, :] = zero
        return carry

    lax.fori_loop(0, _NUM_NESTS, _init_acc, 0)

    def _phase_b(s, sr):
        xs, gs = _pos(s)
        u = plsc.load_gather(w_v, [xs + 17])
        nu = plsc.load_gather(w_v, [gs + 1])
        fi = plsc.load_gather(mark_i, [xs, lanes]) == s
        fn = plsc.load_gather(mark_n, [gs, lanes]) == s
        e = jnp.where(fi, jnp.exp(u - c), jnp.float32(0.0))
        plsc.addupdate_scatter(acc, [gs, lanes], e)
        return sr + jnp.where(fn, jnp.exp(nu - cn), jnp.float32(0.0))

    sr = lax.fori_loop(0, _S, _phase_b, zero)

    ccn = c + cn
    lens = len_v[:]

    def _phase_c(s, carry):
        xs, gs = _pos(s)
        u = plsc.load_gather(w_v, [xs + 17])
        nu = plsc.load_gather(w_v, [gs + 1])
        a = plsc.load_gather(acc, [gs, lanes])
        val = u + nu - ccn - _ln(a * sr)
        val = jnp.where(lens > s, val, neg_inf)
        plsc.store_scatter(out_v, [jnp.broadcast_to(s, (_L,)), lanes], val)
        return carry

    lax.fori_loop(0, _S, _phase_c, 0)

    cps = [pltpu.async_copy(out_v.at[s], out_hbm.at[pl.ds(s * _B + base, _ROWS)],
                            sems.at[3]) for s in range(_S)]
    for cp in cps:
        cp.wait()


@functools.cache
def _build_sc_kernel():
    mesh = plsc.VectorSubcoreMesh(core_axis_name="c", subcore_axis_name="s")
    return pl.kernel(
        _nested_logit_body,
        mesh=mesh,
        compiler_params=pltpu.CompilerParams(needs_layout_passes=False),
        out_type=jax.ShapeDtypeStruct((_B * _S,), jnp.float32),
        scratch_types=[
            pltpu.VMEM((_S, 128), jnp.int32),
            pltpu.VMEM((_ROWS,), jnp.int32),
            pltpu.VMEM((_NUM_NODES + 1,), jnp.float32),
            pltpu.VMEM((_S, _ROWS), jnp.float32),
            pltpu.VMEM((_NUM_ITEMS, _L), jnp.int32),
            pltpu.VMEM((_NUM_NESTS, _L), jnp.int32),
            pltpu.VMEM((_NUM_NESTS, _L), jnp.float32),
            pltpu.SemaphoreType.DMA((4,)),
        ],
    )


def kernel(x, x_lengths, weight, leaf_ancestor_matrix, inf_adjacency_matrix):
    del leaf_ancestor_matrix, inf_adjacency_matrix
    Bn, Sn = x.shape
    out_t = _build_sc_kernel()(x.T, x_lengths, weight.reshape(_NUM_NODES + 1))
    return out_t.reshape(Sn, Bn).T.reshape(Bn, Sn, 1)
